# baseline (speedup 1.0000x reference)
import functools

import jax
import jax.numpy as jnp
from jax import lax
from jax.experimental import pallas as pl

B, S = 1024, 77
GCN_DIM, MODEL_DIM = 512, 768
MAX_OBJS, MAX_SEQ_LEN = 100, 77
MAX_DIST = MAX_OBJS

SEC_A, SEC_B, SEC_C, SEC_D = 208, 104, 208, 8
OFF_SUB_IN_A = 104
T_TOT = SEC_A + SEC_B + SEC_C + SEC_D


BH = 512


def _body(x_ref, tt_ref, oi_ref, sp_ref, op_ref, w1_ref, b1_ref, w2_ref,
          b2_ref, tbl_ref, abs_ref, xc_ref, xm_ref):
    s = pl.program_id(0)
    x = x_ref[0].astype(jnp.bfloat16)
    h = jnp.dot(x, w1_ref[...], preferred_element_type=jnp.float32)
    h = h + b1_ref[...]
    h = 0.5 * h * (1.0 + lax.erf(h * 0.7071067811865476))
    xc = jnp.dot(h.astype(jnp.bfloat16), w2_ref[...],
                 preferred_element_type=jnp.float32) + b2_ref[...]
    xc_ref[0] = xc

    tok = jnp.clip(tt_ref[0], 0, 1)
    oi = jnp.minimum(oi_ref[0], MAX_OBJS - 1)
    sp = jnp.minimum(sp_ref[0], MAX_OBJS - 1)
    op = jnp.minimum(op_ref[0], MAX_OBJS - 1)
    ds = jnp.clip(s - sp, -MAX_DIST, MAX_DIST) + MAX_DIST
    do = jnp.clip(s - op, -MAX_DIST, MAX_DIST) + MAX_DIST
    t0 = tok == 0
    t1 = tok == 1

    neg1 = jnp.full_like(tok, -1)
    idx_a = jnp.where(t0, oi, sp + OFF_SUB_IN_A)
    idx_b = jnp.where(t1, op, neg1)
    idx_c = jnp.where(t1, ds, neg1)
    idx_d = jnp.where(t1, do, neg1)
    ia = lax.broadcasted_iota(jnp.int32, (SEC_A, BH), 0)
    ib = lax.broadcasted_iota(jnp.int32, (SEC_B, BH), 0)
    ic = lax.broadcasted_iota(jnp.int32, (SEC_C, BH), 0)
    it = lax.broadcasted_iota(jnp.int32, (SEC_D, BH), 0)
    ua = (ia == idx_a).astype(jnp.bfloat16)
    ub = (ib == idx_b).astype(jnp.bfloat16)
    uc = ((ic == idx_c).astype(jnp.bfloat16)
          + (ic == idx_d).astype(jnp.bfloat16))
    ud = (it == tok).astype(jnp.bfloat16)
    ut = jnp.concatenate([ua, ub, uc, ud], axis=0)
    emb = lax.dot_general(ut, tbl_ref[...],
                          (((0,), (0,)), ((), ())),
                          preferred_element_type=jnp.float32)
    xm_ref[0] = xc + emb + abs_ref[0]


@functools.partial(jax.jit, static_argnames=())
def kernel(gcn_vectors, token_types, obj_idx, sub_ptr, obj_ptr, W1, b1, W2, b2,
           abs_pos_emb, type_emb, self_idx_emb, sub_ptr_emb, obj_ptr_emb,
           rel_dist_emb):
    z4 = jnp.zeros((4, MODEL_DIM), jnp.float32)
    tbl = jnp.concatenate(
        [self_idx_emb, z4, sub_ptr_emb, z4,
         obj_ptr_emb, z4,
         rel_dist_emb, jnp.zeros((7, MODEL_DIM), jnp.float32),
         type_emb, jnp.zeros((6, MODEL_DIM), jnp.float32)],
        axis=0).astype(jnp.bfloat16)
    xg = jnp.transpose(gcn_vectors, (1, 0, 2))
    tt = jnp.transpose(token_types.astype(jnp.int32)).reshape(S, 1, B)
    oi = jnp.transpose(obj_idx.astype(jnp.int32)).reshape(S, 1, B)
    sp = jnp.transpose(sub_ptr.astype(jnp.int32)).reshape(S, 1, B)
    op = jnp.transpose(obj_ptr.astype(jnp.int32)).reshape(S, 1, B)

    xc, xm = pl.pallas_call(
        _body,
        grid=(S, B // BH),
        in_specs=[
            pl.BlockSpec((1, BH, GCN_DIM), lambda i, j: (i, j, 0)),
            pl.BlockSpec((1, 1, BH), lambda i, j: (i, 0, j)),
            pl.BlockSpec((1, 1, BH), lambda i, j: (i, 0, j)),
            pl.BlockSpec((1, 1, BH), lambda i, j: (i, 0, j)),
            pl.BlockSpec((1, 1, BH), lambda i, j: (i, 0, j)),
            pl.BlockSpec((GCN_DIM, MODEL_DIM), lambda i, j: (0, 0)),
            pl.BlockSpec((1, MODEL_DIM), lambda i, j: (0, 0)),
            pl.BlockSpec((MODEL_DIM, MODEL_DIM), lambda i, j: (0, 0)),
            pl.BlockSpec((1, MODEL_DIM), lambda i, j: (0, 0)),
            pl.BlockSpec((T_TOT, MODEL_DIM), lambda i, j: (0, 0)),
            pl.BlockSpec((1, 1, MODEL_DIM), lambda i, j: (i, 0, 0)),
        ],
        out_specs=[
            pl.BlockSpec((1, BH, MODEL_DIM), lambda i, j: (i, j, 0)),
            pl.BlockSpec((1, BH, MODEL_DIM), lambda i, j: (i, j, 0)),
        ],
        out_shape=[
            jax.ShapeDtypeStruct((S, B, MODEL_DIM), jnp.float32),
            jax.ShapeDtypeStruct((S, B, MODEL_DIM), jnp.float32),
        ],
    )(xg, tt, oi, sp, op,
      W1.astype(jnp.bfloat16), b1.reshape(1, MODEL_DIM),
      W2.astype(jnp.bfloat16), b2.reshape(1, MODEL_DIM), tbl,
      abs_pos_emb.reshape(S, 1, MODEL_DIM))
    return (jnp.transpose(xc, (1, 0, 2)), jnp.transpose(xm, (1, 0, 2)))


# speedup vs baseline: 1.1020x; 1.1020x over previous
import functools

import jax
import jax.numpy as jnp
from jax import lax
from jax.experimental import pallas as pl

B, S = 1024, 77
GCN_DIM, MODEL_DIM = 512, 768
MAX_OBJS, MAX_SEQ_LEN = 100, 77
MAX_DIST = MAX_OBJS

SEC_A, SEC_B, SEC_C, SEC_D = 208, 104, 208, 8
OFF_SUB_IN_A = 104
T_TOT = SEC_A + SEC_B + SEC_C + SEC_D


def _body(x_ref, tt_ref, oi_ref, sp_ref, op_ref, w1_ref, b1_ref, w2_ref,
          b2_ref, tbl_ref, abs_ref, xc_ref, xm_ref):
    s = pl.program_id(0)
    x = x_ref[0].astype(jnp.bfloat16)
    h = jnp.dot(x, w1_ref[...], preferred_element_type=jnp.float32)
    h = h + b1_ref[...]
    h = 0.5 * h * (1.0 + lax.erf(h * 0.7071067811865476))
    xc = jnp.dot(h.astype(jnp.bfloat16), w2_ref[...],
                 preferred_element_type=jnp.float32) + b2_ref[...]
    xc_ref[0] = xc

    tok = jnp.clip(tt_ref[0], 0, 1)
    oi = jnp.minimum(oi_ref[0], MAX_OBJS - 1)
    sp = jnp.minimum(sp_ref[0], MAX_OBJS - 1)
    op = jnp.minimum(op_ref[0], MAX_OBJS - 1)
    ds = jnp.clip(s - sp, -MAX_DIST, MAX_DIST) + MAX_DIST
    do = jnp.clip(s - op, -MAX_DIST, MAX_DIST) + MAX_DIST
    t0 = tok == 0
    t1 = tok == 1

    neg1 = jnp.full_like(tok, -1)
    idx_a = jnp.where(t0, oi, sp + OFF_SUB_IN_A)
    idx_b = jnp.where(t1, op, neg1)
    idx_c = jnp.where(t1, ds, neg1)
    idx_d = jnp.where(t1, do, neg1)
    ia = lax.broadcasted_iota(jnp.int32, (SEC_A, B), 0)
    ib = lax.broadcasted_iota(jnp.int32, (SEC_B, B), 0)
    ic = lax.broadcasted_iota(jnp.int32, (SEC_C, B), 0)
    it = lax.broadcasted_iota(jnp.int32, (SEC_D, B), 0)
    ua = (ia == idx_a).astype(jnp.bfloat16)
    ub = (ib == idx_b).astype(jnp.bfloat16)
    uc = ((ic == idx_c).astype(jnp.bfloat16)
          + (ic == idx_d).astype(jnp.bfloat16))
    ud = (it == tok).astype(jnp.bfloat16)
    ut = jnp.concatenate([ua, ub, uc, ud], axis=0)
    emb = lax.dot_general(ut, tbl_ref[...],
                          (((0,), (0,)), ((), ())),
                          preferred_element_type=jnp.float32)
    xm_ref[0] = xc + emb + abs_ref[0]


@functools.partial(jax.jit, static_argnames=())
def kernel(gcn_vectors, token_types, obj_idx, sub_ptr, obj_ptr, W1, b1, W2, b2,
           abs_pos_emb, type_emb, self_idx_emb, sub_ptr_emb, obj_ptr_emb,
           rel_dist_emb):
    z4 = jnp.zeros((4, MODEL_DIM), jnp.float32)
    tbl = jnp.concatenate(
        [self_idx_emb, z4, sub_ptr_emb, z4,
         obj_ptr_emb, z4,
         rel_dist_emb, jnp.zeros((7, MODEL_DIM), jnp.float32),
         type_emb, jnp.zeros((6, MODEL_DIM), jnp.float32)],
        axis=0).astype(jnp.bfloat16)
    xg = jnp.transpose(gcn_vectors, (1, 0, 2))
    tt = jnp.transpose(token_types.astype(jnp.int32)).reshape(S, 1, B)
    oi = jnp.transpose(obj_idx.astype(jnp.int32)).reshape(S, 1, B)
    sp = jnp.transpose(sub_ptr.astype(jnp.int32)).reshape(S, 1, B)
    op = jnp.transpose(obj_ptr.astype(jnp.int32)).reshape(S, 1, B)

    xc, xm = pl.pallas_call(
        _body,
        grid=(S,),
        in_specs=[
            pl.BlockSpec((1, B, GCN_DIM), lambda i: (i, 0, 0)),
            pl.BlockSpec((1, 1, B), lambda i: (i, 0, 0)),
            pl.BlockSpec((1, 1, B), lambda i: (i, 0, 0)),
            pl.BlockSpec((1, 1, B), lambda i: (i, 0, 0)),
            pl.BlockSpec((1, 1, B), lambda i: (i, 0, 0)),
            pl.BlockSpec((GCN_DIM, MODEL_DIM), lambda i: (0, 0)),
            pl.BlockSpec((1, MODEL_DIM), lambda i: (0, 0)),
            pl.BlockSpec((MODEL_DIM, MODEL_DIM), lambda i: (0, 0)),
            pl.BlockSpec((1, MODEL_DIM), lambda i: (0, 0)),
            pl.BlockSpec((T_TOT, MODEL_DIM), lambda i: (0, 0)),
            pl.BlockSpec((1, 1, MODEL_DIM), lambda i: (i, 0, 0)),
        ],
        out_specs=[
            pl.BlockSpec((1, B, MODEL_DIM), lambda i: (i, 0, 0)),
            pl.BlockSpec((1, B, MODEL_DIM), lambda i: (i, 0, 0)),
        ],
        out_shape=[
            jax.ShapeDtypeStruct((S, B, MODEL_DIM), jnp.float32),
            jax.ShapeDtypeStruct((S, B, MODEL_DIM), jnp.float32),
        ],
    )(xg, tt, oi, sp, op,
      W1.astype(jnp.bfloat16), b1.reshape(1, MODEL_DIM),
      W2.astype(jnp.bfloat16), b2.reshape(1, MODEL_DIM), tbl,
      abs_pos_emb.reshape(S, 1, MODEL_DIM))
    return (jnp.transpose(xc, (1, 0, 2)), jnp.transpose(xm, (1, 0, 2)))
